# baseline (speedup 1.0000x reference)
import functools

import jax
import jax.numpy as jnp
from jax import lax
from jax.experimental import pallas as pl
from jax.experimental.pallas import tpu as pltpu
from jax.experimental.pallas import tpu_sc as plsc

E = 8
TOP_K = 2
DIM = 768
DFF = DIM * 4
T = 2048
NT = T * TOP_K
BT = 256
NTP = NT + E * BT
NBP = NTP // BT
MW = NBP + 8
BTC = 256
SQRT1_2 = 0.7071067811865476


def _router_body(x_ref, Wg_ref, bg_ref,
                 xb_ref, w1_ref, w2_ref, p1_ref, p2_ref, be_ref):
    x = x_ref[...]
    xb_ref[...] = x.astype(jnp.bfloat16)
    logits = jnp.dot(x, Wg_ref[...],
                     preferred_element_type=jnp.float32) + bg_ref[...]
    iota_e = jax.lax.broadcasted_iota(jnp.int32, (T, E), 1)
    m1 = jnp.max(logits, axis=1, keepdims=True)
    a1 = jnp.min(jnp.where(logits == m1, iota_e, E), axis=1, keepdims=True)
    oh1 = iota_e == a1
    l2 = jnp.where(oh1, -jnp.inf, logits)
    m2 = jnp.max(l2, axis=1, keepdims=True)
    a2 = jnp.min(jnp.where(l2 == m2, iota_e, E), axis=1, keepdims=True)
    oh2 = iota_e == a2
    ex2 = jnp.exp(m2 - m1)
    denom = 1.0 + ex2
    w1_ref[...] = 1.0 / denom
    w2_ref[...] = ex2 / denom

    M = (oh1 | oh2).astype(jnp.bfloat16)
    rank_chunks = []
    for rb in range(T // 256):
        row = jax.lax.broadcasted_iota(jnp.int32, (256, T), 0) + rb * 256
        col = jax.lax.broadcasted_iota(jnp.int32, (256, T), 1)
        Lc = (col < row).astype(jnp.bfloat16)
        rank_chunks.append(jnp.dot(Lc, M,
                                   preferred_element_type=jnp.float32))
    ranks = jnp.concatenate(rank_chunks, axis=0)

    last = ranks[T - 1:T, :] + M[T - 1:T, :].astype(jnp.float32)
    offs = 0
    off_list = []
    for k in range(E):
        off_list.append(offs)
        cnt = jnp.sum(last[:, k:k + 1]).astype(jnp.int32)
        offs = offs + ((cnt + BT - 1) // BT) * BT
    blk_base = jax.lax.broadcasted_iota(jnp.int32, (1, MW), 1) * BT
    be = jnp.zeros((1, MW), jnp.int32)
    for k in range(1, E):
        be = be + (blk_base >= off_list[k]).astype(jnp.int32)
    lane_m = jax.lax.broadcasted_iota(jnp.int32, (1, MW), 1)
    be = jnp.where(lane_m == NBP, offs, jnp.where(lane_m > NBP, 0, be))
    be_ref[...] = be

    off_row = jnp.zeros((1, E), jnp.int32)
    off_iota = jax.lax.broadcasted_iota(jnp.int32, (1, E), 1)
    for k in range(E):
        off_row = jnp.where(off_iota == k, off_list[k], off_row)
    pos = ranks.astype(jnp.int32) + off_row
    p1_ref[...] = jnp.sum(jnp.where(oh1, pos, 0), axis=1, keepdims=True)
    p2_ref[...] = jnp.sum(jnp.where(oh2, pos, 0), axis=1, keepdims=True)


def _ffn_body(meta_ref, xb_ref, p1_ref, p2_ref, w1_ref, w2_ref,
              W1_ref, b1_ref, W2_ref, b2_ref, ys_ref):
    b = pl.program_id(0)
    base = b * BT

    @pl.when(base < meta_ref[NBP])
    def _compute():
        lane = jax.lax.broadcasted_iota(jnp.int32, (T, BT), 1) + base
        hit1 = p1_ref[...] == lane
        hit2 = p2_ref[...] == lane
        PT = (hit1 | hit2).astype(jnp.bfloat16)
        wsel = (jnp.where(hit1, w1_ref[...], 0.0)
                + jnp.where(hit2, w2_ref[...], 0.0)).astype(jnp.bfloat16)

        dn = (((0,), (0,)), ((), ()))
        xs = jax.lax.dot_general(PT, xb_ref[...], dn,
                                 preferred_element_type=jnp.float32)
        xs = xs.astype(jnp.bfloat16)
        wrow = jax.lax.dot_general(wsel, jnp.ones((T, 1), jnp.bfloat16), dn,
                                   preferred_element_type=jnp.float32)

        h = jnp.dot(xs, W1_ref[0].astype(jnp.bfloat16),
                    preferred_element_type=jnp.float32) + b1_ref[0]
        h = (h * 0.5 * (1.0 + jax.lax.erf(h * SQRT1_2))).astype(jnp.bfloat16)
        ys = jnp.dot(h, W2_ref[0].astype(jnp.bfloat16),
                     preferred_element_type=jnp.float32) + b2_ref[0]
        ys_ref[...] = ys * wrow

    @pl.when(base >= meta_ref[NBP])
    def _pad():
        ys_ref[...] = jnp.zeros_like(ys_ref)


NWORK = 32
T2 = T // 2
TW = T2 // NWORK


def _tc_combine_body(p1_ref, p2_ref, ys_ref, out_ref, ysb_ref):
    @pl.when(pl.program_id(0) == 0)
    def _cast():
        ysb_ref[...] = ys_ref[...].astype(jnp.bfloat16)

    lane = jax.lax.broadcasted_iota(jnp.int32, (BTC, NTP), 1)
    P = ((p1_ref[...] == lane) | (p2_ref[...] == lane)).astype(jnp.bfloat16)
    out_ref[...] = jnp.dot(P, ysb_ref[...],
                           preferred_element_type=jnp.float32)


def _sc_combine_body(ys_hbm, p1_hbm, p2_hbm, out_hbm,
                     idx1_v, idx2_v, y1_v, y2_v, sem1, sem2):
    wid = lax.axis_index("s") * 2 + lax.axis_index("c")
    base = wid * TW
    pltpu.sync_copy(p1_hbm.at[pl.ds(base, TW)], idx1_v)
    pltpu.sync_copy(p2_hbm.at[pl.ds(base, TW)], idx2_v)
    cp1 = pltpu.async_copy(ys_hbm.at[idx1_v], y1_v, sem1)
    cp2 = pltpu.async_copy(ys_hbm.at[idx2_v], y2_v, sem2)
    cp1.wait()
    cp2.wait()

    def _row(r, _):
        for c in range(DIM // 16):
            sl = pl.ds(c * 16, 16)
            y1_v[r, sl] = y1_v[r, sl] + y2_v[r, sl]
        return 0

    lax.fori_loop(0, TW, _row, 0)
    pltpu.sync_copy(y1_v, out_hbm.at[pl.ds(base, TW)])


_sc_combine = functools.partial(
    pl.kernel,
    out_type=jax.ShapeDtypeStruct((T2, DIM), jnp.float32),
    mesh=plsc.VectorSubcoreMesh(core_axis_name="c", subcore_axis_name="s"),
    scratch_types=[
        pltpu.VMEM((TW,), jnp.int32),
        pltpu.VMEM((TW,), jnp.int32),
        pltpu.VMEM((TW, DIM), jnp.float32),
        pltpu.VMEM((TW, DIM), jnp.float32),
        pltpu.SemaphoreType.DMA,
        pltpu.SemaphoreType.DMA,
    ],
)(_sc_combine_body)


def kernel(x, W1, b1, W2, b2, Wg, bg):
    B, S, _ = x.shape
    x2 = x.reshape(S, DIM)
    bg2 = bg.reshape(1, E)
    b1r = b1.reshape(E, 1, DFF)
    b2r = b2.reshape(E, 1, DIM)

    xb, w1, w2, p1, p2, meta = pl.pallas_call(
        _router_body,
        grid=(1,),
        in_specs=[
            pl.BlockSpec((T, DIM), lambda i: (0, 0)),
            pl.BlockSpec((DIM, E), lambda i: (0, 0)),
            pl.BlockSpec((1, E), lambda i: (0, 0)),
        ],
        out_specs=[
            pl.BlockSpec((T, DIM), lambda i: (0, 0)),
            pl.BlockSpec((T, 1), lambda i: (0, 0)),
            pl.BlockSpec((T, 1), lambda i: (0, 0)),
            pl.BlockSpec((T, 1), lambda i: (0, 0)),
            pl.BlockSpec((T, 1), lambda i: (0, 0)),
            pl.BlockSpec((1, MW), lambda i: (0, 0)),
        ],
        out_shape=[
            jax.ShapeDtypeStruct((T, DIM), jnp.bfloat16),
            jax.ShapeDtypeStruct((T, 1), jnp.float32),
            jax.ShapeDtypeStruct((T, 1), jnp.float32),
            jax.ShapeDtypeStruct((T, 1), jnp.int32),
            jax.ShapeDtypeStruct((T, 1), jnp.int32),
            jax.ShapeDtypeStruct((1, MW), jnp.int32),
        ],
    )(x2, Wg, bg2)

    grid_spec = pltpu.PrefetchScalarGridSpec(
        num_scalar_prefetch=1,
        grid=(NBP,),
        in_specs=[
            pl.BlockSpec((T, DIM), lambda b, m: (0, 0)),
            pl.BlockSpec((T, 1), lambda b, m: (0, 0)),
            pl.BlockSpec((T, 1), lambda b, m: (0, 0)),
            pl.BlockSpec((T, 1), lambda b, m: (0, 0)),
            pl.BlockSpec((T, 1), lambda b, m: (0, 0)),
            pl.BlockSpec((1, DIM, DFF), lambda b, m: (m[b], 0, 0)),
            pl.BlockSpec((1, 1, DFF), lambda b, m: (m[b], 0, 0)),
            pl.BlockSpec((1, DFF, DIM), lambda b, m: (m[b], 0, 0)),
            pl.BlockSpec((1, 1, DIM), lambda b, m: (m[b], 0, 0)),
        ],
        out_specs=pl.BlockSpec((BT, DIM), lambda b, m: (b, 0)),
    )
    ys = pl.pallas_call(
        _ffn_body,
        grid_spec=grid_spec,
        out_shape=jax.ShapeDtypeStruct((NTP, DIM), jnp.float32),
    )(meta.reshape(-1), xb, p1, p2, w1, w2, W1, b1r, W2, b2r)

    out_sc = _sc_combine(ys, p1.reshape(T)[T2:], p2.reshape(T)[T2:])
    out_tc = pl.pallas_call(
        _tc_combine_body,
        grid=(T2 // BTC,),
        in_specs=[
            pl.BlockSpec((BTC, 1), lambda b: (b, 0)),
            pl.BlockSpec((BTC, 1), lambda b: (b, 0)),
            pl.BlockSpec((NTP, DIM), lambda b: (0, 0)),
        ],
        out_specs=pl.BlockSpec((BTC, DIM), lambda b: (b, 0)),
        out_shape=jax.ShapeDtypeStruct((T2, DIM), jnp.float32),
        scratch_shapes=[pltpu.VMEM((NTP, DIM), jnp.bfloat16)],
    )(p1[:T2], p2[:T2], ys)
    out = jnp.concatenate([out_tc, out_sc], axis=0)
    return out.reshape(B, S, DIM)


# speedup vs baseline: 1.1926x; 1.1926x over previous
import functools

import jax
import jax.numpy as jnp
from jax import lax
from jax.experimental import pallas as pl
from jax.experimental.pallas import tpu as pltpu
from jax.experimental.pallas import tpu_sc as plsc

E = 8
TOP_K = 2
DIM = 768
DFF = DIM * 4
T = 2048
NT = T * TOP_K
BT = 512
NTP = NT + E * BT
NBP = NTP // BT
MW = NBP + 8
BTC = 256
SQRT1_2 = 0.7071067811865476


def _router_body(x_ref, Wg_ref, bg_ref,
                 xb_ref, w1_ref, w2_ref, p1_ref, p2_ref, be_ref):
    x = x_ref[...]
    xb_ref[...] = x.astype(jnp.bfloat16)
    logits = jnp.dot(x, Wg_ref[...],
                     preferred_element_type=jnp.float32) + bg_ref[...]
    iota_e = jax.lax.broadcasted_iota(jnp.int32, (T, E), 1)
    m1 = jnp.max(logits, axis=1, keepdims=True)
    a1 = jnp.min(jnp.where(logits == m1, iota_e, E), axis=1, keepdims=True)
    oh1 = iota_e == a1
    l2 = jnp.where(oh1, -jnp.inf, logits)
    m2 = jnp.max(l2, axis=1, keepdims=True)
    a2 = jnp.min(jnp.where(l2 == m2, iota_e, E), axis=1, keepdims=True)
    oh2 = iota_e == a2
    ex2 = jnp.exp(m2 - m1)
    denom = 1.0 + ex2
    w1_ref[...] = 1.0 / denom
    w2_ref[...] = ex2 / denom

    M = (oh1 | oh2).astype(jnp.bfloat16)
    rank_chunks = []
    for rb in range(T // 256):
        row = jax.lax.broadcasted_iota(jnp.int32, (256, T), 0) + rb * 256
        col = jax.lax.broadcasted_iota(jnp.int32, (256, T), 1)
        Lc = (col < row).astype(jnp.bfloat16)
        rank_chunks.append(jnp.dot(Lc, M,
                                   preferred_element_type=jnp.float32))
    ranks = jnp.concatenate(rank_chunks, axis=0)

    last = ranks[T - 1:T, :] + M[T - 1:T, :].astype(jnp.float32)
    offs = 0
    off_list = []
    for k in range(E):
        off_list.append(offs)
        cnt = jnp.sum(last[:, k:k + 1]).astype(jnp.int32)
        offs = offs + ((cnt + BT - 1) // BT) * BT
    blk_base = jax.lax.broadcasted_iota(jnp.int32, (1, MW), 1) * BT
    be = jnp.zeros((1, MW), jnp.int32)
    for k in range(1, E):
        be = be + (blk_base >= off_list[k]).astype(jnp.int32)
    lane_m = jax.lax.broadcasted_iota(jnp.int32, (1, MW), 1)
    be = jnp.where(lane_m == NBP, offs, jnp.where(lane_m > NBP, 0, be))
    be_ref[...] = be

    off_row = jnp.zeros((1, E), jnp.int32)
    off_iota = jax.lax.broadcasted_iota(jnp.int32, (1, E), 1)
    for k in range(E):
        off_row = jnp.where(off_iota == k, off_list[k], off_row)
    pos = ranks.astype(jnp.int32) + off_row
    p1_ref[...] = jnp.sum(jnp.where(oh1, pos, 0), axis=1, keepdims=True)
    p2_ref[...] = jnp.sum(jnp.where(oh2, pos, 0), axis=1, keepdims=True)


def _ffn_body(meta_ref, xb_ref, p1_ref, p2_ref, w1_ref, w2_ref,
              W1_ref, b1_ref, W2_ref, b2_ref, ys_ref):
    b = pl.program_id(0)
    base = b * BT

    @pl.when(base < meta_ref[NBP])
    def _compute():
        lane = jax.lax.broadcasted_iota(jnp.int32, (T, BT), 1) + base
        hit1 = p1_ref[...] == lane
        hit2 = p2_ref[...] == lane
        PT = (hit1 | hit2).astype(jnp.bfloat16)
        wsel = (jnp.where(hit1, w1_ref[...], 0.0)
                + jnp.where(hit2, w2_ref[...], 0.0)).astype(jnp.bfloat16)

        dn = (((0,), (0,)), ((), ()))
        xs = jax.lax.dot_general(PT, xb_ref[...], dn,
                                 preferred_element_type=jnp.float32)
        xs = xs.astype(jnp.bfloat16)
        wrow = jax.lax.dot_general(wsel, jnp.ones((T, 1), jnp.bfloat16), dn,
                                   preferred_element_type=jnp.float32)

        h = jnp.dot(xs, W1_ref[0].astype(jnp.bfloat16),
                    preferred_element_type=jnp.float32) + b1_ref[0]
        h = (h * 0.5 * (1.0 + jax.lax.erf(h * SQRT1_2))).astype(jnp.bfloat16)
        ys = jnp.dot(h, W2_ref[0].astype(jnp.bfloat16),
                     preferred_element_type=jnp.float32) + b2_ref[0]
        ys_ref[...] = ys * wrow

    @pl.when(base >= meta_ref[NBP])
    def _pad():
        ys_ref[...] = jnp.zeros_like(ys_ref)


NWORK = 32
TW = T // NWORK


def _sc_combine_body(ys_hbm, p1_hbm, p2_hbm, out_hbm,
                     idx1_v, idx2_v, y1_v, y2_v, sem1, sem2):
    wid = lax.axis_index("s") * 2 + lax.axis_index("c")
    base = wid * TW
    pltpu.sync_copy(p1_hbm.at[pl.ds(base, TW)], idx1_v)
    pltpu.sync_copy(p2_hbm.at[pl.ds(base, TW)], idx2_v)
    cp1 = pltpu.async_copy(ys_hbm.at[idx1_v], y1_v, sem1)
    cp2 = pltpu.async_copy(ys_hbm.at[idx2_v], y2_v, sem2)
    cp1.wait()
    cp2.wait()

    def _row(r, _):
        for c in range(DIM // 16):
            sl = pl.ds(c * 16, 16)
            y1_v[r, sl] = y1_v[r, sl] + y2_v[r, sl]
        return 0

    lax.fori_loop(0, TW, _row, 0)
    pltpu.sync_copy(y1_v, out_hbm.at[pl.ds(base, TW)])


_sc_combine = functools.partial(
    pl.kernel,
    out_type=jax.ShapeDtypeStruct((T, DIM), jnp.float32),
    mesh=plsc.VectorSubcoreMesh(core_axis_name="c", subcore_axis_name="s"),
    scratch_types=[
        pltpu.VMEM((TW,), jnp.int32),
        pltpu.VMEM((TW,), jnp.int32),
        pltpu.VMEM((TW, DIM), jnp.float32),
        pltpu.VMEM((TW, DIM), jnp.float32),
        pltpu.SemaphoreType.DMA,
        pltpu.SemaphoreType.DMA,
    ],
)(_sc_combine_body)


def kernel(x, W1, b1, W2, b2, Wg, bg):
    B, S, _ = x.shape
    x2 = x.reshape(S, DIM)
    bg2 = bg.reshape(1, E)
    b1r = b1.reshape(E, 1, DFF)
    b2r = b2.reshape(E, 1, DIM)

    xb, w1, w2, p1, p2, meta = pl.pallas_call(
        _router_body,
        grid=(1,),
        in_specs=[
            pl.BlockSpec((T, DIM), lambda i: (0, 0)),
            pl.BlockSpec((DIM, E), lambda i: (0, 0)),
            pl.BlockSpec((1, E), lambda i: (0, 0)),
        ],
        out_specs=[
            pl.BlockSpec((T, DIM), lambda i: (0, 0)),
            pl.BlockSpec((T, 1), lambda i: (0, 0)),
            pl.BlockSpec((T, 1), lambda i: (0, 0)),
            pl.BlockSpec((T, 1), lambda i: (0, 0)),
            pl.BlockSpec((T, 1), lambda i: (0, 0)),
            pl.BlockSpec((1, MW), lambda i: (0, 0)),
        ],
        out_shape=[
            jax.ShapeDtypeStruct((T, DIM), jnp.bfloat16),
            jax.ShapeDtypeStruct((T, 1), jnp.float32),
            jax.ShapeDtypeStruct((T, 1), jnp.float32),
            jax.ShapeDtypeStruct((T, 1), jnp.int32),
            jax.ShapeDtypeStruct((T, 1), jnp.int32),
            jax.ShapeDtypeStruct((1, MW), jnp.int32),
        ],
    )(x2, Wg, bg2)

    grid_spec = pltpu.PrefetchScalarGridSpec(
        num_scalar_prefetch=1,
        grid=(NBP,),
        in_specs=[
            pl.BlockSpec((T, DIM), lambda b, m: (0, 0)),
            pl.BlockSpec((T, 1), lambda b, m: (0, 0)),
            pl.BlockSpec((T, 1), lambda b, m: (0, 0)),
            pl.BlockSpec((T, 1), lambda b, m: (0, 0)),
            pl.BlockSpec((T, 1), lambda b, m: (0, 0)),
            pl.BlockSpec((1, DIM, DFF), lambda b, m: (m[b], 0, 0)),
            pl.BlockSpec((1, 1, DFF), lambda b, m: (m[b], 0, 0)),
            pl.BlockSpec((1, DFF, DIM), lambda b, m: (m[b], 0, 0)),
            pl.BlockSpec((1, 1, DIM), lambda b, m: (m[b], 0, 0)),
        ],
        out_specs=pl.BlockSpec((BT, DIM), lambda b, m: (b, 0)),
    )
    ys = pl.pallas_call(
        _ffn_body,
        grid_spec=grid_spec,
        out_shape=jax.ShapeDtypeStruct((NTP, DIM), jnp.float32),
    )(meta.reshape(-1), xb, p1, p2, w1, w2, W1, b1r, W2, b2r)

    out = _sc_combine(ys, p1.reshape(T), p2.reshape(T))
    return out.reshape(B, S, DIM)
